# baseline (speedup 1.0000x reference)
import functools

import jax
import jax.numpy as jnp
from jax import lax
from jax.experimental import pallas as pl
from jax.experimental.pallas import tpu as pltpu
from jax.experimental.pallas import tpu_sc as plsc

TOP_K = 2
G = 256


def _router_body(x_ref, gw_ref, gb_ref, logits_ref, idx_ref, wn_ref):
    x = x_ref[...]
    logits = lax.dot_general(
        x, gw_ref[...], (((1,), (1,)), ((), ())),
        preferred_element_type=jnp.float32) + gb_ref[...]
    logits_ref[...] = logits
    m1 = jnp.max(logits, axis=-1, keepdims=True)
    a1 = jnp.argmax(logits, axis=-1)
    cols = lax.broadcasted_iota(jnp.int32, logits.shape, 1)
    logits2 = jnp.where(cols == a1[:, None], -jnp.inf, logits)
    m2 = jnp.max(logits2, axis=-1, keepdims=True)
    a2 = jnp.argmax(logits2, axis=-1)
    w1 = 1.0 / (1.0 + jnp.exp(m2 - m1))
    idx_ref[...] = jnp.concatenate(
        [a1[:, None], a2[:, None]], axis=1).astype(jnp.int32)
    wn_ref[...] = jnp.concatenate([w1, 1.0 - w1], axis=1)


def _router(xs, gate_W, gate_b):
    T, H = xs.shape
    E = gate_W.shape[0]
    TM = 1024
    return pl.pallas_call(
        _router_body,
        grid=(T // TM,),
        in_specs=[
            pl.BlockSpec((TM, H), lambda i: (i, 0)),
            pl.BlockSpec((E, H), lambda i: (0, 0)),
            pl.BlockSpec((1, E), lambda i: (0, 0)),
        ],
        out_specs=[
            pl.BlockSpec((TM, E), lambda i: (i, 0)),
            pl.BlockSpec((TM, TOP_K), lambda i: (i, 0)),
            pl.BlockSpec((TM, TOP_K), lambda i: (i, 0)),
        ],
        out_shape=[
            jax.ShapeDtypeStruct((T, E), jnp.float32),
            jax.ShapeDtypeStruct((T, TOP_K), jnp.int32),
            jax.ShapeDtypeStruct((T, TOP_K), jnp.float32),
        ],
    )(xs, gate_W, gate_b.reshape(1, E))


def _make_gather(T, W32, P):
    info = plsc.get_sparse_core_info()
    NC, NS = info.num_cores, info.num_subcores
    NW = NC * NS
    rows_w = P // NW
    CH = 64
    n_ch = rows_w // CH
    mesh = plsc.VectorSubcoreMesh(core_axis_name="c", subcore_axis_name="s")

    @functools.partial(
        pl.kernel, mesh=mesh,
        out_type=jax.ShapeDtypeStruct((P, W32), jnp.int32),
        scratch_types=[
            pltpu.VMEM((CH,), jnp.int32),
            pltpu.VMEM((CH, W32), jnp.int32),
            pltpu.SemaphoreType.DMA,
        ],
    )
    def gather(x_hbm, src_hbm, a_hbm, idx_v, rows_v, sem):
        wid = lax.axis_index("s") * NC + lax.axis_index("c")
        base = pl.multiple_of(wid * rows_w, CH)
        for j in range(n_ch):
            b = base + j * CH
            pltpu.sync_copy(src_hbm.at[pl.ds(b, CH)], idx_v)
            pltpu.async_copy(x_hbm.at[idx_v], rows_v, sem).wait()
            pltpu.sync_copy(rows_v, a_hbm.at[pl.ds(b, CH)])

    return gather


def _gmm_body(te_ref, a_ref, w_ref, eb_ref, wr_ref, y_ref):
    acc = lax.dot_general(
        a_ref[...], w_ref[0], (((1,), (1,)), ((), ())),
        preferred_element_type=jnp.float32)
    y_ref[...] = (acc + eb_ref[0]) * wr_ref[...]


def _gmm(te, A_bf, ewb, eb3, w_rows, P, H, E):
    NT = P // G
    grid_spec = pltpu.PrefetchScalarGridSpec(
        num_scalar_prefetch=1,
        grid=(NT,),
        in_specs=[
            pl.BlockSpec((G, H), lambda i, te: (i, 0)),
            pl.BlockSpec((1, H, H), lambda i, te: (te[i], 0, 0)),
            pl.BlockSpec((1, 1, H), lambda i, te: (te[i], 0, 0)),
            pl.BlockSpec((G, 1), lambda i, te: (i, 0)),
        ],
        out_specs=pl.BlockSpec((G, H), lambda i, te: (i, 0)),
    )
    return pl.pallas_call(
        _gmm_body,
        grid_spec=grid_spec,
        out_shape=jax.ShapeDtypeStruct((P, H), jnp.float32),
    )(te, A_bf, ewb, eb3, w_rows)


def _make_combine(T, H, P):
    info = plsc.get_sparse_core_info()
    NC, NS = info.num_cores, info.num_subcores
    NW = NC * NS
    tok_w = T // NW
    CH = 16
    n_ch = tok_w // CH
    mesh = plsc.VectorSubcoreMesh(core_axis_name="c", subcore_axis_name="s")

    @functools.partial(
        pl.kernel, mesh=mesh,
        out_type=jax.ShapeDtypeStruct((T, H), jnp.float32),
        scratch_types=[
            pltpu.VMEM((CH,), jnp.int32),
            pltpu.VMEM((CH,), jnp.int32),
            pltpu.VMEM((CH, H), jnp.float32),
            pltpu.VMEM((CH, H), jnp.float32),
            pltpu.VMEM((CH, H), jnp.float32),
            pltpu.SemaphoreType.DMA,
            pltpu.SemaphoreType.DMA,
        ],
    )
    def combine(y_hbm, d0_hbm, d1_hbm, out_hbm,
                i0_v, i1_v, y0_v, y1_v, o_v, sem0, sem1):
        wid = lax.axis_index("s") * NC + lax.axis_index("c")
        base = pl.multiple_of(wid * tok_w, CH)
        ng = H // 16

        def _chunk(j, carry):
            b = pl.multiple_of(base + j * CH, CH)
            pltpu.sync_copy(d0_hbm.at[pl.ds(b, CH)], i0_v)
            pltpu.sync_copy(d1_hbm.at[pl.ds(b, CH)], i1_v)
            c0 = pltpu.async_copy(y_hbm.at[i0_v], y0_v, sem0)
            c1 = pltpu.async_copy(y_hbm.at[i1_v], y1_v, sem1)
            c0.wait()
            c1.wait()

            def _row(r, carry2):
                def _add(g, carry3):
                    s = pl.ds(g * 16, 16)
                    o_v[r, s] = y0_v[r, s] + y1_v[r, s]
                    return carry3
                return lax.fori_loop(0, ng, _add, carry2, unroll=4)

            lax.fori_loop(0, CH, _row, 0)
            pltpu.sync_copy(o_v, out_hbm.at[pl.ds(b, CH)])
            return carry

        lax.fori_loop(0, n_ch, _chunk, 0)

    return combine


def kernel(x, gate_W, gate_b, expert_W, expert_b):
    batch, seq, H = x.shape
    E = gate_W.shape[0]
    T = batch * seq
    P = 2 * T + E * G
    NT = P // G
    xs = x.reshape(T, H)

    logits, idx, wn = _router(xs, gate_W, gate_b)

    e_all = idx.reshape(2 * T)
    oh = (e_all[:, None] == jnp.arange(E, dtype=jnp.int32)[None, :]).astype(jnp.int32)
    cum = jnp.cumsum(oh, axis=0)
    counts = cum[-1]
    rank = jnp.sum(cum * oh, axis=1) - 1
    padded = ((counts + G - 1) // G) * G
    starts = jnp.concatenate([jnp.zeros((1,), jnp.int32),
                              jnp.cumsum(padded)])[:E]
    dest = starts[e_all] + rank
    tile_start = jnp.arange(NT, dtype=jnp.int32) * G
    ends = starts + padded
    te = jnp.minimum(jnp.sum(tile_start[:, None] >= ends[None, :], axis=1),
                     E - 1).astype(jnp.int32)
    w_flat = wn.reshape(2 * T)
    w_rows = jnp.zeros((P,), jnp.float32).at[dest].set(w_flat)
    src = jnp.zeros((P,), jnp.int32).at[dest].set(
        jnp.arange(2 * T, dtype=jnp.int32) // 2)
    dpair = dest.reshape(T, 2)
    d0 = dpair[:, 0]
    d1 = dpair[:, 1]

    W32 = H // 2
    xb32 = lax.bitcast_convert_type(
        xs.astype(jnp.bfloat16).reshape(T, W32, 2), jnp.int32)
    A32 = _make_gather(T, W32, P)(xb32, src)
    A_bf = lax.bitcast_convert_type(A32, jnp.bfloat16).reshape(P, H)

    Y = _gmm(te, A_bf, expert_W.astype(jnp.bfloat16),
             expert_b.reshape(E, 1, H), w_rows.reshape(P, 1), P, H, E)

    out = _make_combine(T, H, P)(Y, d0, d1)
    return out.reshape(batch, seq, H), logits


# speedup vs baseline: 3.3349x; 3.3349x over previous
import functools

import jax
import jax.numpy as jnp
from jax import lax
from jax.experimental import pallas as pl
from jax.experimental.pallas import tpu as pltpu
from jax.experimental.pallas import tpu_sc as plsc

TOP_K = 2
G = 256


def _router_body(x_ref, gw_ref, gb_ref, logits_ref, idx_ref, wn_ref):
    x = x_ref[...]
    logits = lax.dot_general(
        x, gw_ref[...], (((1,), (1,)), ((), ())),
        preferred_element_type=jnp.float32) + gb_ref[...]
    logits_ref[...] = logits
    m1 = jnp.max(logits, axis=-1, keepdims=True)
    a1 = jnp.argmax(logits, axis=-1)
    cols = lax.broadcasted_iota(jnp.int32, logits.shape, 1)
    logits2 = jnp.where(cols == a1[:, None], -jnp.inf, logits)
    m2 = jnp.max(logits2, axis=-1, keepdims=True)
    a2 = jnp.argmax(logits2, axis=-1)
    w1 = 1.0 / (1.0 + jnp.exp(m2 - m1))
    idx_ref[...] = jnp.concatenate(
        [a1[:, None], a2[:, None]], axis=1).astype(jnp.int32)
    wn_ref[...] = jnp.concatenate([w1, 1.0 - w1], axis=1)


def _router(xs, gate_W, gate_b):
    T, H = xs.shape
    E = gate_W.shape[0]
    TM = 1024
    return pl.pallas_call(
        _router_body,
        grid=(T // TM,),
        in_specs=[
            pl.BlockSpec((TM, H), lambda i: (i, 0)),
            pl.BlockSpec((E, H), lambda i: (0, 0)),
            pl.BlockSpec((1, E), lambda i: (0, 0)),
        ],
        out_specs=[
            pl.BlockSpec((TM, E), lambda i: (i, 0)),
            pl.BlockSpec((TM, TOP_K), lambda i: (i, 0)),
            pl.BlockSpec((TM, TOP_K), lambda i: (i, 0)),
        ],
        out_shape=[
            jax.ShapeDtypeStruct((T, E), jnp.float32),
            jax.ShapeDtypeStruct((T, TOP_K), jnp.int32),
            jax.ShapeDtypeStruct((T, TOP_K), jnp.float32),
        ],
    )(xs, gate_W, gate_b.reshape(1, E))


def _make_dispatch(T, H, P):
    info = plsc.get_sparse_core_info()
    NC, NS = info.num_cores, info.num_subcores
    NW = NC * NS
    tok_w = T // NW
    CH = 16
    n_ch = tok_w // CH
    mesh = plsc.VectorSubcoreMesh(core_axis_name="c", subcore_axis_name="s")

    @functools.partial(
        pl.kernel, mesh=mesh,
        out_type=jax.ShapeDtypeStruct((P, H), jnp.float32),
        scratch_types=[
            pltpu.VMEM((CH, H), jnp.float32),
            pltpu.VMEM((CH,), jnp.int32),
            pltpu.VMEM((CH,), jnp.int32),
            pltpu.SemaphoreType.DMA,
            pltpu.SemaphoreType.DMA,
        ],
    )
    def dispatch(x_hbm, d0_hbm, d1_hbm, a_hbm, xv, i0_v, i1_v, sem0, sem1):
        wid = lax.axis_index("s") * NC + lax.axis_index("c")
        base = pl.multiple_of(wid * tok_w, CH)

        def _chunk(j, carry):
            b = pl.multiple_of(base + j * CH, CH)
            pltpu.sync_copy(x_hbm.at[pl.ds(b, CH)], xv)
            pltpu.sync_copy(d0_hbm.at[pl.ds(b, CH)], i0_v)
            pltpu.sync_copy(d1_hbm.at[pl.ds(b, CH)], i1_v)
            s0 = pltpu.async_copy(xv, a_hbm.at[i0_v], sem0)
            s1 = pltpu.async_copy(xv, a_hbm.at[i1_v], sem1)
            s0.wait()
            s1.wait()
            return carry

        lax.fori_loop(0, n_ch, _chunk, 0)

    return dispatch


def _gmm_body(te_ref, a_ref, w_ref, eb_ref, y_ref):
    acc = lax.dot_general(
        a_ref[...], w_ref[0], (((1,), (1,)), ((), ())),
        preferred_element_type=jnp.float32)
    y_ref[...] = acc + eb_ref[0]


def _gmm(te, A, eW, eb3, P, H, E):
    NT = P // G
    grid_spec = pltpu.PrefetchScalarGridSpec(
        num_scalar_prefetch=1,
        grid=(NT,),
        in_specs=[
            pl.BlockSpec((G, H), lambda i, te: (i, 0)),
            pl.BlockSpec((1, H, H), lambda i, te: (te[i], 0, 0)),
            pl.BlockSpec((1, 1, H), lambda i, te: (te[i], 0, 0)),
        ],
        out_specs=pl.BlockSpec((G, H), lambda i, te: (i, 0)),
    )
    return pl.pallas_call(
        _gmm_body,
        grid_spec=grid_spec,
        out_shape=jax.ShapeDtypeStruct((P, H), jnp.float32),
    )(te, A, eW, eb3)


def _make_combine(T, H, P):
    info = plsc.get_sparse_core_info()
    NC, NS = info.num_cores, info.num_subcores
    NW = NC * NS
    tok_w = T // NW
    CH = 16
    n_ch = tok_w // CH
    mesh = plsc.VectorSubcoreMesh(core_axis_name="c", subcore_axis_name="s")

    @functools.partial(
        pl.kernel, mesh=mesh,
        out_type=jax.ShapeDtypeStruct((T, H), jnp.float32),
        scratch_types=[
            pltpu.VMEM((CH,), jnp.int32),
            pltpu.VMEM((CH,), jnp.int32),
            pltpu.VMEM((CH, 16), jnp.float32),
            pltpu.VMEM((CH, 16), jnp.float32),
            pltpu.VMEM((CH, H), jnp.float32),
            pltpu.VMEM((CH, H), jnp.float32),
            pltpu.VMEM((CH, H), jnp.float32),
            pltpu.SemaphoreType.DMA,
            pltpu.SemaphoreType.DMA,
        ],
    )
    def combine(y_hbm, d0_hbm, d1_hbm, w0_hbm, w1_hbm, out_hbm,
                i0_v, i1_v, w0_v, w1_v, y0_v, y1_v, o_v, sem0, sem1):
        wid = lax.axis_index("s") * NC + lax.axis_index("c")
        base = pl.multiple_of(wid * tok_w, CH)
        ng = H // 16

        def _chunk(j, carry):
            b = pl.multiple_of(base + j * CH, CH)
            pltpu.sync_copy(d0_hbm.at[pl.ds(b, CH)], i0_v)
            pltpu.sync_copy(d1_hbm.at[pl.ds(b, CH)], i1_v)
            pltpu.sync_copy(w0_hbm.at[pl.ds(b, CH)], w0_v)
            pltpu.sync_copy(w1_hbm.at[pl.ds(b, CH)], w1_v)
            c0 = pltpu.async_copy(y_hbm.at[i0_v], y0_v, sem0)
            c1 = pltpu.async_copy(y_hbm.at[i1_v], y1_v, sem1)
            c0.wait()
            c1.wait()

            def _row(r, carry2):
                wb0 = w0_v[r, pl.ds(0, 16)]
                wb1 = w1_v[r, pl.ds(0, 16)]

                def _add(g, carry3):
                    s = pl.ds(g * 16, 16)
                    o_v[r, s] = wb0 * y0_v[r, s] + wb1 * y1_v[r, s]
                    return carry3
                return lax.fori_loop(0, ng, _add, carry2, unroll=4)

            lax.fori_loop(0, CH, _row, 0)
            pltpu.sync_copy(o_v, out_hbm.at[pl.ds(b, CH)])
            return carry

        lax.fori_loop(0, n_ch, _chunk, 0)

    return combine


def kernel(x, gate_W, gate_b, expert_W, expert_b):
    batch, seq, H = x.shape
    E = gate_W.shape[0]
    T = batch * seq
    P = 2 * T + E * G
    NT = P // G
    xs = x.reshape(T, H)

    logits, idx, wn = _router(xs, gate_W, gate_b)

    e_all = idx.reshape(2 * T)
    oh = (e_all[:, None] == jnp.arange(E, dtype=jnp.int32)[None, :]).astype(jnp.int32)
    cum = jnp.cumsum(oh, axis=0)
    counts = cum[-1]
    rank = jnp.sum(cum * oh, axis=1) - 1
    padded = ((counts + G - 1) // G) * G
    starts = jnp.concatenate([jnp.zeros((1,), jnp.int32),
                              jnp.cumsum(padded)])[:E]
    dest = starts[e_all] + rank
    tile_start = jnp.arange(NT, dtype=jnp.int32) * G
    ends = starts + padded
    te = jnp.minimum(jnp.sum(tile_start[:, None] >= ends[None, :], axis=1),
                     E - 1).astype(jnp.int32)
    dpair = dest.reshape(T, 2)
    d0 = dpair[:, 0]
    d1 = dpair[:, 1]
    w0x = jnp.broadcast_to(wn[:, 0:1], (T, 16))
    w1x = jnp.broadcast_to(wn[:, 1:2], (T, 16))

    A = _make_dispatch(T, H, P)(xs, d0, d1)

    Y = _gmm(te, A, expert_W, expert_b.reshape(E, 1, H), P, H, E)

    out = _make_combine(T, H, P)(Y, d0, d1, w0x, w1x)
    return out.reshape(batch, seq, H), logits


# speedup vs baseline: 3.5085x; 1.0521x over previous
import functools

import jax
import jax.numpy as jnp
from jax import lax
from jax.experimental import pallas as pl
from jax.experimental.pallas import tpu as pltpu
from jax.experimental.pallas import tpu_sc as plsc

TOP_K = 2
G = 256


def _router_body(x_ref, gw_ref, gb_ref, logits_ref, idx_ref, wn_ref):
    x = x_ref[...]
    logits = lax.dot_general(
        x, gw_ref[...], (((1,), (1,)), ((), ())),
        preferred_element_type=jnp.float32) + gb_ref[...]
    logits_ref[...] = logits
    m1 = jnp.max(logits, axis=-1, keepdims=True)
    a1 = jnp.argmax(logits, axis=-1)
    cols = lax.broadcasted_iota(jnp.int32, logits.shape, 1)
    logits2 = jnp.where(cols == a1[:, None], -jnp.inf, logits)
    m2 = jnp.max(logits2, axis=-1, keepdims=True)
    a2 = jnp.argmax(logits2, axis=-1)
    w1 = 1.0 / (1.0 + jnp.exp(m2 - m1))
    idx_ref[...] = jnp.concatenate(
        [a1[:, None], a2[:, None]], axis=1).astype(jnp.int32)
    wn_ref[...] = jnp.concatenate([w1, 1.0 - w1], axis=1)


def _router(xs, gate_W, gate_b):
    T, H = xs.shape
    E = gate_W.shape[0]
    TM = 1024
    return pl.pallas_call(
        _router_body,
        grid=(T // TM,),
        in_specs=[
            pl.BlockSpec((TM, H), lambda i: (i, 0)),
            pl.BlockSpec((E, H), lambda i: (0, 0)),
            pl.BlockSpec((1, E), lambda i: (0, 0)),
        ],
        out_specs=[
            pl.BlockSpec((TM, E), lambda i: (i, 0)),
            pl.BlockSpec((TM, TOP_K), lambda i: (i, 0)),
            pl.BlockSpec((TM, TOP_K), lambda i: (i, 0)),
        ],
        out_shape=[
            jax.ShapeDtypeStruct((T, E), jnp.float32),
            jax.ShapeDtypeStruct((T, TOP_K), jnp.int32),
            jax.ShapeDtypeStruct((T, TOP_K), jnp.float32),
        ],
    )(xs, gate_W, gate_b.reshape(1, E))


def _make_dispatch(T, H, P):
    info = plsc.get_sparse_core_info()
    NC, NS = info.num_cores, info.num_subcores
    NW = NC * NS
    tok_w = T // NW
    CH = 16
    n_ch = tok_w // CH
    mesh = plsc.VectorSubcoreMesh(core_axis_name="c", subcore_axis_name="s")

    @functools.partial(
        pl.kernel, mesh=mesh,
        out_type=jax.ShapeDtypeStruct((P, H), jnp.float32),
        scratch_types=[
            pltpu.VMEM((CH, H), jnp.float32),
            pltpu.VMEM((CH, H), jnp.float32),
            pltpu.VMEM((CH,), jnp.int32),
            pltpu.VMEM((CH,), jnp.int32),
            pltpu.VMEM((CH,), jnp.int32),
            pltpu.VMEM((CH,), jnp.int32),
            pltpu.SemaphoreType.DMA,
            pltpu.SemaphoreType.DMA,
            pltpu.SemaphoreType.DMA,
            pltpu.SemaphoreType.DMA,
        ],
    )
    def dispatch(x_hbm, d0_hbm, d1_hbm, a_hbm,
                 xva, xvb, i0a, i0b, i1a, i1b, s0a, s0b, s1a, s1b):
        wid = lax.axis_index("s") * NC + lax.axis_index("c")
        base = pl.multiple_of(wid * tok_w, CH)
        bufs = ((xva, i0a, i1a, s0a, s1a), (xvb, i0b, i1b, s0b, s1b))

        def _pair(j2, carry):
            for b in (0, 1):
                xv, i0_v, i1_v, sem0, sem1 = bufs[b]
                j = j2 * 2 + b

                @pl.when(j2 > 0)
                def _drain():
                    pltpu.make_async_copy(xv, a_hbm.at[i0_v], sem0).wait()
                    pltpu.make_async_copy(xv, a_hbm.at[i1_v], sem1).wait()

                bb = pl.multiple_of(base + j * CH, CH)
                pltpu.sync_copy(x_hbm.at[pl.ds(bb, CH)], xv)
                pltpu.sync_copy(d0_hbm.at[pl.ds(bb, CH)], i0_v)
                pltpu.sync_copy(d1_hbm.at[pl.ds(bb, CH)], i1_v)
                pltpu.async_copy(xv, a_hbm.at[i0_v], sem0)
                pltpu.async_copy(xv, a_hbm.at[i1_v], sem1)
            return carry

        lax.fori_loop(0, n_ch // 2, _pair, 0)
        for b in (0, 1):
            xv, i0_v, i1_v, sem0, sem1 = bufs[b]
            pltpu.make_async_copy(xv, a_hbm.at[i0_v], sem0).wait()
            pltpu.make_async_copy(xv, a_hbm.at[i1_v], sem1).wait()

    return dispatch


def _gmm_body(te_ref, a_ref, w_ref, eb_ref, y_ref):
    acc = lax.dot_general(
        a_ref[...], w_ref[0], (((1,), (1,)), ((), ())),
        preferred_element_type=jnp.float32)
    y_ref[...] = acc + eb_ref[0]


def _gmm(te, A, eW, eb3, P, H, E):
    NT = P // G
    grid_spec = pltpu.PrefetchScalarGridSpec(
        num_scalar_prefetch=1,
        grid=(NT,),
        in_specs=[
            pl.BlockSpec((G, H), lambda i, te: (i, 0)),
            pl.BlockSpec((1, H, H), lambda i, te: (te[i], 0, 0)),
            pl.BlockSpec((1, 1, H), lambda i, te: (te[i], 0, 0)),
        ],
        out_specs=pl.BlockSpec((G, H), lambda i, te: (i, 0)),
    )
    return pl.pallas_call(
        _gmm_body,
        grid_spec=grid_spec,
        out_shape=jax.ShapeDtypeStruct((P, H), jnp.float32),
    )(te, A, eW, eb3)


def _make_combine(T, H, P):
    info = plsc.get_sparse_core_info()
    NC, NS = info.num_cores, info.num_subcores
    NW = NC * NS
    tok_w = T // NW
    CH = 8
    n_ch = tok_w // CH
    mesh = plsc.VectorSubcoreMesh(core_axis_name="c", subcore_axis_name="s")

    buf = lambda *s: [pltpu.VMEM(s, jnp.float32) for _ in range(2)]

    @functools.partial(
        pl.kernel, mesh=mesh,
        out_type=jax.ShapeDtypeStruct((T, H), jnp.float32),
        scratch_types=(
            [pltpu.VMEM((CH,), jnp.int32) for _ in range(4)]
            + buf(CH, 16) + buf(CH, 16) + buf(CH, H) + buf(CH, H) + buf(CH, H)
            + [pltpu.SemaphoreType.DMA for _ in range(4)]
        ),
    )
    def combine(y_hbm, d0_hbm, d1_hbm, w0_hbm, w1_hbm, out_hbm,
                i0a, i0b, i1a, i1b, w0a, w0b, w1a, w1b,
                y0a, y0b, y1a, y1b, oa, ob, s0a, s0b, s1a, s1b):
        wid = lax.axis_index("s") * NC + lax.axis_index("c")
        base = pl.multiple_of(wid * tok_w, CH)
        ng = H // 16
        bufs = ((i0a, i1a, w0a, w1a, y0a, y1a, oa, s0a, s1a),
                (i0b, i1b, w0b, w1b, y0b, y1b, ob, s0b, s1b))

        def _load(j, b):
            i0_v, i1_v, w0_v, w1_v, y0_v, y1_v, o_v, sem0, sem1 = bufs[b]
            bb = pl.multiple_of(base + j * CH, CH)
            pltpu.sync_copy(d0_hbm.at[pl.ds(bb, CH)], i0_v)
            pltpu.sync_copy(d1_hbm.at[pl.ds(bb, CH)], i1_v)
            pltpu.sync_copy(w0_hbm.at[pl.ds(bb, CH)], w0_v)
            pltpu.sync_copy(w1_hbm.at[pl.ds(bb, CH)], w1_v)
            pltpu.async_copy(y_hbm.at[i0_v], y0_v, sem0)
            pltpu.async_copy(y_hbm.at[i1_v], y1_v, sem1)

        for b in (0, 1):
            _load(b, b)

        def _pair(j2, carry):
            for b in (0, 1):
                i0_v, i1_v, w0_v, w1_v, y0_v, y1_v, o_v, sem0, sem1 = bufs[b]
                j = j2 * 2 + b
                pltpu.make_async_copy(y_hbm.at[i0_v], y0_v, sem0).wait()
                pltpu.make_async_copy(y_hbm.at[i1_v], y1_v, sem1).wait()

                def _row(r, carry2):
                    wb0 = w0_v[r, pl.ds(0, 16)]
                    wb1 = w1_v[r, pl.ds(0, 16)]

                    def _add(g, carry3):
                        s = pl.ds(g * 16, 16)
                        o_v[r, s] = wb0 * y0_v[r, s] + wb1 * y1_v[r, s]
                        return carry3
                    return lax.fori_loop(0, ng, _add, carry2, unroll=4)

                lax.fori_loop(0, CH, _row, 0)
                bb = pl.multiple_of(base + j * CH, CH)
                pltpu.sync_copy(o_v, out_hbm.at[pl.ds(bb, CH)])

                @pl.when(j + 2 < n_ch)
                def _next():
                    _load(j + 2, b)
            return carry

        lax.fori_loop(0, n_ch // 2, _pair, 0)

    return combine


def kernel(x, gate_W, gate_b, expert_W, expert_b):
    batch, seq, H = x.shape
    E = gate_W.shape[0]
    T = batch * seq
    P = 2 * T + E * G
    NT = P // G
    xs = x.reshape(T, H)

    logits, idx, wn = _router(xs, gate_W, gate_b)

    e_all = idx.reshape(2 * T)
    oh = (e_all[:, None] == jnp.arange(E, dtype=jnp.int32)[None, :]).astype(jnp.int32)
    cum = jnp.cumsum(oh, axis=0)
    counts = cum[-1]
    rank = jnp.sum(cum * oh, axis=1) - 1
    padded = ((counts + G - 1) // G) * G
    starts = jnp.concatenate([jnp.zeros((1,), jnp.int32),
                              jnp.cumsum(padded)])[:E]
    dest = starts[e_all] + rank
    tile_start = jnp.arange(NT, dtype=jnp.int32) * G
    ends = starts + padded
    te = jnp.minimum(jnp.sum(tile_start[:, None] >= ends[None, :], axis=1),
                     E - 1).astype(jnp.int32)
    dpair = dest.reshape(T, 2)
    d0 = dpair[:, 0]
    d1 = dpair[:, 1]
    w0x = jnp.broadcast_to(wn[:, 0:1], (T, 16))
    w1x = jnp.broadcast_to(wn[:, 1:2], (T, 16))

    A = _make_dispatch(T, H, P)(xs, d0, d1)

    Y = _gmm(te, A, expert_W, expert_b.reshape(E, 1, H), P, H, E)

    out = _make_combine(T, H, P)(Y, d0, d1, w0x, w1x)
    return out.reshape(batch, seq, H), logits
